# Initial kernel scaffold; baseline (speedup 1.0000x reference)
import jax
import jax.numpy as jnp
from jax.experimental import pallas as pl


def kernel(x, edge_index, graph_ids, self_feat, x3d, W1, b1, W2, b2, Wq2, Wk2, Wv2, Wo2, ln2_g, ln2_b, Wq3, Wk3, Wv3, Wo3, ln3_g, ln3_b, fc1_W, fc1_b, fc2_W, fc2_b):
    raise NotImplementedError("write your pallas kernel here")


# speedup vs baseline: 4.9157x; 4.9157x over previous
import jax
import jax.numpy as jnp
from jax import lax
from jax.experimental import pallas as pl
from jax.experimental.pallas import tpu as pltpu
from jax.experimental.pallas import tpu_sc as plsc

N_NODES = 10000
ROWS = 10240
NC, NS = 2, 16
NW = NC * NS
CH = 128
K = 80
EPAD = NW * K * CH
RPT = ROWS // NS


def _sc_agg(table, srcb, dstb, z2, z1, d, with_deg):
  mesh = plsc.VectorSubcoreMesh(core_axis_name="c", subcore_axis_name="s")
  outs = [jax.ShapeDtypeStruct((NC, ROWS, d), jnp.float32)]
  scratch = [
      pltpu.VMEM((K, CH), jnp.int32),
      pltpu.VMEM((K, CH), jnp.int32),
      pltpu.VMEM((CH, d), jnp.float32),
      pltpu.VMEM_SHARED((ROWS, d), jnp.float32),
      pltpu.SemaphoreType.DMA,
  ]
  if with_deg:
    outs.append(jax.ShapeDtypeStruct((NC, ROWS), jnp.float32))
    scratch += [pltpu.VMEM((CH,), jnp.float32),
                pltpu.VMEM_SHARED((ROWS,), jnp.float32)]

  def body(table_h, src_h, dst_h, z2_h, z1_h, *rest):
    if with_deg:
      acc_out, deg_out, sidx, didx, rows, acc_sh, sem, ones, deg_sh = rest
    else:
      acc_out, sidx, didx, rows, acc_sh, sem = rest
    cid = lax.axis_index("c")
    sid = lax.axis_index("s")
    wid = sid * NC + cid
    r0 = sid * RPT
    pltpu.sync_copy(z2_h.at[pl.ds(r0, RPT)], acc_sh.at[pl.ds(r0, RPT)])
    pltpu.sync_copy(src_h.at[wid], sidx)
    pltpu.sync_copy(dst_h.at[wid], didx)
    if with_deg:
      pltpu.sync_copy(z1_h.at[pl.ds(r0, RPT)], deg_sh.at[pl.ds(r0, RPT)])
      for i in range(CH // 16):
        ones[pl.ds(i * 16, 16)] = jnp.full((16,), 1.0, jnp.float32)
    plsc.subcore_barrier()

    def step(k, carry):
      pltpu.async_copy(table_h.at[sidx.at[k]], rows, sem).wait()
      pltpu.sync_copy(rows, acc_sh.at[didx.at[k]], add=True)
      if with_deg:
        pltpu.sync_copy(ones, deg_sh.at[didx.at[k]], add=True)
      return carry
    lax.fori_loop(0, K, step, 0)

    plsc.subcore_barrier()
    pltpu.sync_copy(acc_sh.at[pl.ds(r0, RPT)], acc_out.at[cid, pl.ds(r0, RPT)])
    if with_deg:
      pltpu.sync_copy(deg_sh.at[pl.ds(r0, RPT)], deg_out.at[cid, pl.ds(r0, RPT)])

  f = pl.kernel(body, out_type=outs, mesh=mesh, scratch_types=scratch,
                compiler_params=pltpu.CompilerParams(use_tc_tiling_on_sc=False))
  return f(table, srcb, dstb, z2, z1)


def _tc_mid(acc, degb, W1, b1, W2):
  def body(acc_ref, deg_ref, w1_ref, b1_ref, w2_ref, out_ref):
    a = acc_ref[0] + acc_ref[1]
    mean = a / jnp.maximum(deg_ref[...], 1.0)
    h1 = jnp.maximum(
        lax.dot_general(mean, w1_ref[...], (((1,), (1,)), ((), ())),
                        preferred_element_type=jnp.float32)
        + b1_ref[...][None, :], 0.0)
    y2 = lax.dot_general(h1, w2_ref[...], (((1,), (1,)), ((), ())),
                         preferred_element_type=jnp.float32)
    out_ref[...] = jnp.concatenate(
        [y2, jnp.zeros((ROWS, 12), jnp.float32)], axis=1)
  return pl.pallas_call(
      body, out_shape=jax.ShapeDtypeStruct((ROWS, 32), jnp.float32),
  )(acc, degb, W1, b1, W2)


def _tc_head(acc2, degb2, b2p, gid, self_feat, x3d, Wv2, Wo2, g2, bb2,
             Wv3, Wo3, g3, bb3, f1w, f1b, f2w, f2b):
  def body(acc_ref, deg_ref, b2_ref, gid_ref, sf_ref, x3_ref, wv2_ref,
           wo2_ref, g2_ref, bb2_ref, wv3_ref, wo3_ref, g3_ref, bb3_ref,
           f1w_ref, f1b_ref, f2w_ref, f2b_ref, out_ref):
    acc = acc_ref[0] + acc_ref[1]
    mean2 = acc / jnp.maximum(deg_ref[...], 1.0)
    h2 = jnp.maximum(mean2[:N_NODES] + b2_ref[...][None, :], 0.0)
    gid_v = gid_ref[...]
    iot = lax.broadcasted_iota(jnp.int32, (128, N_NODES), 0)
    mask = (iot == gid_v[None, :]).astype(jnp.float32)
    cnt = jnp.sum(mask, axis=1, keepdims=True)
    pooled = lax.dot_general(mask, h2, (((1,), (0,)), ((), ())),
                             preferred_element_type=jnp.float32)
    hg = pooled[:, :20] / jnp.maximum(cnt, 1.0)

    def ln(t, g, b):
      m = jnp.mean(t, axis=1, keepdims=True)
      v = jnp.mean((t - m) ** 2, axis=1, keepdims=True)
      return (t - m) / jnp.sqrt(v + 1e-5) * g[None, :] + b[None, :]

    def vwo(inp, wv, wo):
      v = lax.dot_general(inp, wv, (((1,), (1,)), ((), ())),
                          preferred_element_type=jnp.float32)
      return lax.dot_general(v, wo, (((1,), (1,)), ((), ())),
                             preferred_element_type=jnp.float32)

    t1 = ln(hg + vwo(sf_ref[...], wv2_ref[...], wo2_ref[...]),
            g2_ref[...], bb2_ref[...])
    t2 = ln(t1 + vwo(x3_ref[...], wv3_ref[...], wo3_ref[...]),
            g3_ref[...], bb3_ref[...])
    f = jnp.maximum(
        lax.dot_general(t2, f1w_ref[...], (((1,), (1,)), ((), ())),
                        preferred_element_type=jnp.float32)
        + f1b_ref[...][None, :], 0.0)
    out_ref[...] = jnp.sum(f * f2w_ref[...], axis=1,
                           keepdims=True) + f2b_ref[0]
  return pl.pallas_call(
      body, out_shape=jax.ShapeDtypeStruct((128, 1), jnp.float32),
  )(acc2, degb2, b2p, gid, self_feat, x3d, Wv2, Wo2, g2, bb2,
    Wv3, Wo3, g3, bb3, f1w, f1b, f2w, f2b)


def kernel(x, edge_index, graph_ids, self_feat, x3d, W1, b1, W2, b2,
           Wq2, Wk2, Wv2, Wo2, ln2_g, ln2_b,
           Wq3, Wk3, Wv3, Wo3, ln3_g, ln3_b,
           fc1_W, fc1_b, fc2_W, fc2_b):
  src = edge_index[0].astype(jnp.int32)
  dst = edge_index[1].astype(jnp.int32)
  e = src.shape[0]
  pad = EPAD - e
  srcp = jnp.concatenate([src, jnp.zeros((pad,), jnp.int32)]).reshape(NW, K, CH)
  dstp = jnp.concatenate(
      [dst, jnp.full((pad,), N_NODES, jnp.int32)]).reshape(NW, K, CH)
  z128 = jnp.zeros((ROWS, 128), jnp.float32)
  z32 = jnp.zeros((ROWS, 32), jnp.float32)
  z1 = jnp.zeros((ROWS,), jnp.float32)

  acc1, deg = _sc_agg(x, srcp, dstp, z128, z1, 128, True)
  degs = deg[0] + deg[1]
  degb = jnp.broadcast_to(degs[:, None], (ROWS, 128))
  y2 = _tc_mid(acc1, degb, W1, b1, W2)
  acc2 = _sc_agg(y2, srcp, dstp, z32, z1, 32, False)[0]
  degb2 = jnp.broadcast_to(degs[:, None], (ROWS, 32))
  b2p = jnp.concatenate([b2, jnp.zeros((12,), jnp.float32)])
  return _tc_head(acc2, degb2, b2p, graph_ids.astype(jnp.int32),
                  self_feat, x3d, Wv2, Wo2, ln2_g, ln2_b,
                  Wv3, Wo3, ln3_g, ln3_b, fc1_W, fc1_b, fc2_W, fc2_b)


# speedup vs baseline: 6.3698x; 1.2958x over previous
import jax
import jax.numpy as jnp
from jax import lax
from jax.experimental import pallas as pl
from jax.experimental.pallas import tpu as pltpu
from jax.experimental.pallas import tpu_sc as plsc

N_NODES = 10000
ROWS = 10240
NC, NS = 2, 16
NW = NC * NS
CH = 128
K = 80
EPAD = NW * K * CH
RPT = ROWS // NS


def _sc_agg(table, srcb, dstb, z2, z1, d, with_deg):
  mesh = plsc.VectorSubcoreMesh(core_axis_name="c", subcore_axis_name="s")
  ibk = 16 if d > 32 else K
  kb = K // ibk
  outs = [jax.ShapeDtypeStruct((NC, ROWS, d), jnp.float32)]
  scratch = [
      pltpu.VMEM((ibk, CH), jnp.int32),
      pltpu.VMEM((ibk, CH), jnp.int32),
      pltpu.VMEM((CH, d), jnp.float32),
      pltpu.VMEM((CH, d), jnp.float32),
      pltpu.VMEM_SHARED((ROWS, d), jnp.float32),
      pltpu.SemaphoreType.DMA,
      pltpu.SemaphoreType.DMA,
  ]
  if with_deg:
    outs.append(jax.ShapeDtypeStruct((NC, ROWS), jnp.float32))
    scratch += [pltpu.VMEM((CH,), jnp.float32),
                pltpu.VMEM_SHARED((ROWS,), jnp.float32)]

  def body(table_h, src_h, dst_h, z2_h, z1_h, *rest):
    if with_deg:
      (acc_out, deg_out, sidx, didx, rows0, rows1, acc_sh, sem0, sem1,
       ones, deg_sh) = rest
    else:
      acc_out, sidx, didx, rows0, rows1, acc_sh, sem0, sem1 = rest
    cid = lax.axis_index("c")
    sid = lax.axis_index("s")
    wid = sid * NC + cid
    r0 = sid * RPT
    pltpu.sync_copy(z2_h.at[pl.ds(r0, RPT)], acc_sh.at[pl.ds(r0, RPT)])
    if with_deg:
      pltpu.sync_copy(z1_h.at[pl.ds(r0, RPT)], deg_sh.at[pl.ds(r0, RPT)])
      for i in range(CH // 16):
        ones[pl.ds(i * 16, 16)] = jnp.full((16,), 1.0, jnp.float32)
    plsc.subcore_barrier()

    def drain(k, rows, sem):
      pltpu.make_async_copy(table_h.at[sidx.at[k]], rows, sem).wait()

    def scat(k, rows):
      pltpu.sync_copy(rows, acc_sh.at[didx.at[k]], add=True)
      if with_deg:
        pltpu.sync_copy(ones, deg_sh.at[didx.at[k]], add=True)

    NI = ibk // 2

    def block(blk, carry):
      pltpu.sync_copy(src_h.at[wid, pl.ds(blk * ibk, ibk)], sidx)
      pltpu.sync_copy(dst_h.at[wid, pl.ds(blk * ibk, ibk)], didx)
      pltpu.async_copy(table_h.at[sidx.at[0]], rows0, sem0)

      def step(i, c):
        k0 = 2 * i
        k1 = k0 + 1
        pltpu.async_copy(table_h.at[sidx.at[k1]], rows1, sem1)
        drain(k0, rows0, sem0)
        scat(k0, rows0)

        @pl.when(i + 1 < NI)
        def _():
          pltpu.async_copy(table_h.at[sidx.at[k0 + 2]], rows0, sem0)
        drain(k1, rows1, sem1)
        scat(k1, rows1)
        return c
      lax.fori_loop(0, NI, step, 0)
      return carry
    lax.fori_loop(0, kb, block, 0)

    plsc.subcore_barrier()
    pltpu.sync_copy(acc_sh.at[pl.ds(r0, RPT)], acc_out.at[cid, pl.ds(r0, RPT)])
    if with_deg:
      pltpu.sync_copy(deg_sh.at[pl.ds(r0, RPT)], deg_out.at[cid, pl.ds(r0, RPT)])

  f = pl.kernel(body, out_type=outs, mesh=mesh, scratch_types=scratch,
                compiler_params=pltpu.CompilerParams(use_tc_tiling_on_sc=False))
  return f(table, srcb, dstb, z2, z1)


def _tc_mid(acc, degb, W1, b1, W2):
  def body(acc_ref, deg_ref, w1_ref, b1_ref, w2_ref, out_ref):
    a = acc_ref[0] + acc_ref[1]
    mean = a / jnp.maximum(deg_ref[...], 1.0)
    h1 = jnp.maximum(
        lax.dot_general(mean, w1_ref[...], (((1,), (1,)), ((), ())),
                        preferred_element_type=jnp.float32)
        + b1_ref[...][None, :], 0.0)
    y2 = lax.dot_general(h1, w2_ref[...], (((1,), (1,)), ((), ())),
                         preferred_element_type=jnp.float32)
    out_ref[...] = jnp.concatenate(
        [y2, jnp.zeros((ROWS, 12), jnp.float32)], axis=1)
  return pl.pallas_call(
      body, out_shape=jax.ShapeDtypeStruct((ROWS, 32), jnp.float32),
  )(acc, degb, W1, b1, W2)


def _tc_head(acc2, degb2, b2p, gid, self_feat, x3d, Wv2, Wo2, g2, bb2,
             Wv3, Wo3, g3, bb3, f1w, f1b, f2w, f2b):
  def body(acc_ref, deg_ref, b2_ref, gid_ref, sf_ref, x3_ref, wv2_ref,
           wo2_ref, g2_ref, bb2_ref, wv3_ref, wo3_ref, g3_ref, bb3_ref,
           f1w_ref, f1b_ref, f2w_ref, f2b_ref, out_ref):
    acc = acc_ref[0] + acc_ref[1]
    mean2 = acc / jnp.maximum(deg_ref[...], 1.0)
    h2 = jnp.maximum(mean2[:N_NODES] + b2_ref[...][None, :], 0.0)
    gid_v = gid_ref[...]
    iot = lax.broadcasted_iota(jnp.int32, (128, N_NODES), 0)
    mask = (iot == gid_v[None, :]).astype(jnp.float32)
    cnt = jnp.sum(mask, axis=1, keepdims=True)
    pooled = lax.dot_general(mask, h2, (((1,), (0,)), ((), ())),
                             preferred_element_type=jnp.float32)
    hg = pooled[:, :20] / jnp.maximum(cnt, 1.0)

    def ln(t, g, b):
      m = jnp.mean(t, axis=1, keepdims=True)
      v = jnp.mean((t - m) ** 2, axis=1, keepdims=True)
      return (t - m) / jnp.sqrt(v + 1e-5) * g[None, :] + b[None, :]

    def vwo(inp, wv, wo):
      v = lax.dot_general(inp, wv, (((1,), (1,)), ((), ())),
                          preferred_element_type=jnp.float32)
      return lax.dot_general(v, wo, (((1,), (1,)), ((), ())),
                             preferred_element_type=jnp.float32)

    t1 = ln(hg + vwo(sf_ref[...], wv2_ref[...], wo2_ref[...]),
            g2_ref[...], bb2_ref[...])
    t2 = ln(t1 + vwo(x3_ref[...], wv3_ref[...], wo3_ref[...]),
            g3_ref[...], bb3_ref[...])
    f = jnp.maximum(
        lax.dot_general(t2, f1w_ref[...], (((1,), (1,)), ((), ())),
                        preferred_element_type=jnp.float32)
        + f1b_ref[...][None, :], 0.0)
    out_ref[...] = jnp.sum(f * f2w_ref[...], axis=1,
                           keepdims=True) + f2b_ref[0]
  return pl.pallas_call(
      body, out_shape=jax.ShapeDtypeStruct((128, 1), jnp.float32),
  )(acc2, degb2, b2p, gid, self_feat, x3d, Wv2, Wo2, g2, bb2,
    Wv3, Wo3, g3, bb3, f1w, f1b, f2w, f2b)


def kernel(x, edge_index, graph_ids, self_feat, x3d, W1, b1, W2, b2,
           Wq2, Wk2, Wv2, Wo2, ln2_g, ln2_b,
           Wq3, Wk3, Wv3, Wo3, ln3_g, ln3_b,
           fc1_W, fc1_b, fc2_W, fc2_b):
  src = edge_index[0].astype(jnp.int32)
  dst = edge_index[1].astype(jnp.int32)
  e = src.shape[0]
  pad = EPAD - e
  srcp = jnp.concatenate([src, jnp.zeros((pad,), jnp.int32)]).reshape(NW, K, CH)
  dstp = jnp.concatenate(
      [dst, jnp.full((pad,), N_NODES, jnp.int32)]).reshape(NW, K, CH)
  z128 = jnp.zeros((ROWS, 128), jnp.float32)
  z32 = jnp.zeros((ROWS, 32), jnp.float32)
  z1 = jnp.zeros((ROWS,), jnp.float32)

  acc1, deg = _sc_agg(x, srcp, dstp, z128, z1, 128, True)
  degs = deg[0] + deg[1]
  degb = jnp.broadcast_to(degs[:, None], (ROWS, 128))
  y2 = _tc_mid(acc1, degb, W1, b1, W2)
  acc2 = _sc_agg(y2, srcp, dstp, z32, z1, 32, False)[0]
  degb2 = jnp.broadcast_to(degs[:, None], (ROWS, 32))
  b2p = jnp.concatenate([b2, jnp.zeros((12,), jnp.float32)])
  return _tc_head(acc2, degb2, b2p, graph_ids.astype(jnp.int32),
                  self_feat, x3d, Wv2, Wo2, ln2_g, ln2_b,
                  Wv3, Wo3, ln3_g, ln3_b, fc1_W, fc1_b, fc2_W, fc2_b)


# speedup vs baseline: 11.2901x; 1.7725x over previous
import jax
import jax.numpy as jnp
from jax import lax
from jax.experimental import pallas as pl
from jax.experimental.pallas import tpu as pltpu
from jax.experimental.pallas import tpu_sc as plsc

N_NODES = 10000
ROWS = 10240
NC, NS = 2, 16
NW = NC * NS
CH = 96
K = 105
EPAD = NW * K * CH
RPT = ROWS // NS


def _sc_agg(table, srcb, dstb, z2, z1, d, with_deg):
  mesh = plsc.VectorSubcoreMesh(core_axis_name="c", subcore_axis_name="s")
  ibk = 15 if d > 32 else K
  kb = K // ibk
  outs = [jax.ShapeDtypeStruct((NC, ROWS, d), jnp.float32)]
  scratch = [
      pltpu.VMEM((ibk, CH), jnp.int32),
      pltpu.VMEM((ibk, CH), jnp.int32),
      pltpu.VMEM((CH, d), jnp.float32),
      pltpu.VMEM((CH, d), jnp.float32),
      pltpu.VMEM((CH, d), jnp.float32),
      pltpu.VMEM_SHARED((ROWS, d), jnp.float32),
      pltpu.SemaphoreType.DMA,
      pltpu.SemaphoreType.DMA,
      pltpu.SemaphoreType.DMA,
      pltpu.SemaphoreType.DMA,
      pltpu.SemaphoreType.DMA,
      pltpu.SemaphoreType.DMA,
      pltpu.SemaphoreType.DMA,
  ]
  if with_deg:
    outs.append(jax.ShapeDtypeStruct((NC, ROWS), jnp.float32))
    scratch += [pltpu.VMEM((CH,), jnp.float32),
                pltpu.VMEM_SHARED((ROWS,), jnp.float32)]

  def body(table_h, src_h, dst_h, z2_h, z1_h, *rest):
    ones = deg_sh = deg_out = None
    if with_deg:
      (acc_out, deg_out, sidx, didx, rb0, rb1, rb2, acc_sh,
       sg0, sg1, sg2, ss0, ss1, ss2, semd, ones, deg_sh) = rest
    else:
      (acc_out, sidx, didx, rb0, rb1, rb2, acc_sh,
       sg0, sg1, sg2, ss0, ss1, ss2, semd) = rest
    rows = (rb0, rb1, rb2)
    semg = (sg0, sg1, sg2)
    sems = (ss0, ss1, ss2)
    cid = lax.axis_index("c")
    sid = lax.axis_index("s")
    wid = sid * NC + cid
    r0 = sid * RPT
    pltpu.sync_copy(z2_h.at[pl.ds(r0, RPT)], acc_sh.at[pl.ds(r0, RPT)])
    if with_deg:
      pltpu.sync_copy(z1_h.at[pl.ds(r0, RPT)], deg_sh.at[pl.ds(r0, RPT)])
      for i in range(CH // 16):
        ones[pl.ds(i * 16, 16)] = jnp.full((16,), 1.0, jnp.float32)
    plsc.subcore_barrier()

    def wait_g(k, b):
      pltpu.make_async_copy(table_h.at[sidx.at[k]], rows[b], semg[b]).wait()

    def wait_s(b):
      pltpu.make_async_copy(rows[b], acc_sh.at[didx.at[0]], sems[b]).wait()

    def wait_d():
      pltpu.make_async_copy(ones, deg_sh.at[didx.at[0]], semd).wait()

    def block(blk, carry):
      pltpu.sync_copy(src_h.at[wid, pl.ds(blk * ibk, ibk)], sidx)
      pltpu.sync_copy(dst_h.at[wid, pl.ds(blk * ibk, ibk)], didx)
      pltpu.async_copy(table_h.at[sidx.at[0]], rows[0], semg[0])
      pltpu.async_copy(table_h.at[sidx.at[1]], rows[1], semg[1])

      def step(j, c):
        for b in range(3):
          k = 3 * j + b
          wait_g(k, b)
          pltpu.async_copy(rows[b], acc_sh.at[didx.at[k]], sems[b], add=True)
          if with_deg:
            @pl.when(k >= 1)
            def _():
              wait_d()
            pltpu.async_copy(ones, deg_sh.at[didx.at[k]], semd, add=True)

          @pl.when(jnp.logical_and(k >= 1, k + 2 < ibk))
          def _():
            wait_s((b + 2) % 3)

          @pl.when(k + 2 < ibk)
          def _():
            pltpu.async_copy(
                table_h.at[sidx.at[k + 2]], rows[(b + 2) % 3],
                semg[(b + 2) % 3])
        return c
      lax.fori_loop(0, ibk // 3, step, 0)
      for b in range(3):
        wait_s(b)
      if with_deg:
        wait_d()
      return carry
    lax.fori_loop(0, kb, block, 0)

    plsc.subcore_barrier()
    pltpu.sync_copy(acc_sh.at[pl.ds(r0, RPT)], acc_out.at[cid, pl.ds(r0, RPT)])
    if with_deg:
      pltpu.sync_copy(deg_sh.at[pl.ds(r0, RPT)], deg_out.at[cid, pl.ds(r0, RPT)])

  f = pl.kernel(body, out_type=outs, mesh=mesh, scratch_types=scratch,
                compiler_params=pltpu.CompilerParams(use_tc_tiling_on_sc=False))
  return f(table, srcb, dstb, z2, z1)


def _tc_mid(acc, degb, W1, b1, W2):
  def body(acc_ref, deg_ref, w1_ref, b1_ref, w2_ref, out_ref):
    a = acc_ref[0] + acc_ref[1]
    mean = a / jnp.maximum(deg_ref[...], 1.0)
    h1 = jnp.maximum(
        lax.dot_general(mean, w1_ref[...], (((1,), (1,)), ((), ())),
                        preferred_element_type=jnp.float32)
        + b1_ref[...][None, :], 0.0)
    y2 = lax.dot_general(h1, w2_ref[...], (((1,), (1,)), ((), ())),
                         preferred_element_type=jnp.float32)
    out_ref[...] = jnp.concatenate(
        [y2, jnp.zeros((ROWS, 12), jnp.float32)], axis=1)
  return pl.pallas_call(
      body, out_shape=jax.ShapeDtypeStruct((ROWS, 32), jnp.float32),
  )(acc, degb, W1, b1, W2)


def _tc_head(acc2, degb2, b2p, gid, self_feat, x3d, Wv2, Wo2, g2, bb2,
             Wv3, Wo3, g3, bb3, f1w, f1b, f2w, f2b):
  def body(acc_ref, deg_ref, b2_ref, gid_ref, sf_ref, x3_ref, wv2_ref,
           wo2_ref, g2_ref, bb2_ref, wv3_ref, wo3_ref, g3_ref, bb3_ref,
           f1w_ref, f1b_ref, f2w_ref, f2b_ref, out_ref):
    acc = acc_ref[0] + acc_ref[1]
    mean2 = acc / jnp.maximum(deg_ref[...], 1.0)
    h2 = jnp.maximum(mean2[:N_NODES] + b2_ref[...][None, :], 0.0)
    gid_v = gid_ref[...]
    iot = lax.broadcasted_iota(jnp.int32, (128, N_NODES), 0)
    mask = (iot == gid_v[None, :]).astype(jnp.float32)
    cnt = jnp.sum(mask, axis=1, keepdims=True)
    pooled = lax.dot_general(mask, h2, (((1,), (0,)), ((), ())),
                             preferred_element_type=jnp.float32)
    hg = pooled[:, :20] / jnp.maximum(cnt, 1.0)

    def ln(t, g, b):
      m = jnp.mean(t, axis=1, keepdims=True)
      v = jnp.mean((t - m) ** 2, axis=1, keepdims=True)
      return (t - m) / jnp.sqrt(v + 1e-5) * g[None, :] + b[None, :]

    def vwo(inp, wv, wo):
      v = lax.dot_general(inp, wv, (((1,), (1,)), ((), ())),
                          preferred_element_type=jnp.float32)
      return lax.dot_general(v, wo, (((1,), (1,)), ((), ())),
                             preferred_element_type=jnp.float32)

    t1 = ln(hg + vwo(sf_ref[...], wv2_ref[...], wo2_ref[...]),
            g2_ref[...], bb2_ref[...])
    t2 = ln(t1 + vwo(x3_ref[...], wv3_ref[...], wo3_ref[...]),
            g3_ref[...], bb3_ref[...])
    f = jnp.maximum(
        lax.dot_general(t2, f1w_ref[...], (((1,), (1,)), ((), ())),
                        preferred_element_type=jnp.float32)
        + f1b_ref[...][None, :], 0.0)
    out_ref[...] = jnp.sum(f * f2w_ref[...], axis=1,
                           keepdims=True) + f2b_ref[0]
  return pl.pallas_call(
      body, out_shape=jax.ShapeDtypeStruct((128, 1), jnp.float32),
  )(acc2, degb2, b2p, gid, self_feat, x3d, Wv2, Wo2, g2, bb2,
    Wv3, Wo3, g3, bb3, f1w, f1b, f2w, f2b)


def kernel(x, edge_index, graph_ids, self_feat, x3d, W1, b1, W2, b2,
           Wq2, Wk2, Wv2, Wo2, ln2_g, ln2_b,
           Wq3, Wk3, Wv3, Wo3, ln3_g, ln3_b,
           fc1_W, fc1_b, fc2_W, fc2_b):
  src = edge_index[0].astype(jnp.int32)
  dst = edge_index[1].astype(jnp.int32)
  e = src.shape[0]
  pad = EPAD - e
  srcp = jnp.concatenate([src, jnp.zeros((pad,), jnp.int32)]).reshape(NW, K, CH)
  dstp = jnp.concatenate(
      [dst, jnp.full((pad,), N_NODES, jnp.int32)]).reshape(NW, K, CH)
  z128 = jnp.zeros((ROWS, 128), jnp.float32)
  z32 = jnp.zeros((ROWS, 32), jnp.float32)
  z1 = jnp.zeros((ROWS,), jnp.float32)

  acc1, deg = _sc_agg(x, srcp, dstp, z128, z1, 128, True)
  degs = deg[0] + deg[1]
  degb = jnp.broadcast_to(degs[:, None], (ROWS, 128))
  y2 = _tc_mid(acc1, degb, W1, b1, W2)
  acc2 = _sc_agg(y2, srcp, dstp, z32, z1, 32, False)[0]
  degb2 = jnp.broadcast_to(degs[:, None], (ROWS, 32))
  b2p = jnp.concatenate([b2, jnp.zeros((12,), jnp.float32)])
  return _tc_head(acc2, degb2, b2p, graph_ids.astype(jnp.int32),
                  self_feat, x3d, Wv2, Wo2, ln2_g, ln2_b,
                  Wv3, Wo3, ln3_g, ln3_b, fc1_W, fc1_b, fc2_W, fc2_b)
